# baseline (speedup 1.0000x reference)
import functools

import jax
import jax.numpy as jnp
from jax import lax
from jax.experimental import pallas as pl
from jax.experimental.pallas import tpu as pltpu
from jax.experimental.pallas import tpu_sc as plsc

NC, NS, L = 2, 16, 16
NW = NC * NS


def _sc_body(T, B, D, CHUNK,
             tef_hbm, t_hbm, dt_hbm, y_hbm, yn_hbm, idx_hbm, out_hbm,
             idx_v, t_v, dt_v, th_v, eq_v, tef_v, yh_v, ynh_v, vals_v,
             zsem, ssem):
    P = CHUNK // 2
    HB = P // 2
    W = 2 * D
    wid = lax.axis_index("s") * NC + lax.axis_index("c")
    base = pl.multiple_of(wid * CHUNK, CHUNK)
    pb = pl.multiple_of(wid * (CHUNK // 2), CHUNK // 2)

    zrow = jnp.zeros((L,), jnp.float32)

    def zbody(i, _):
        for k in range(W // L):
            vals_v[i, pl.ds(k * L, L)] = zrow
        return 0
    lax.fori_loop(0, P, zbody, 0)

    zcopies = [
        pltpu.make_async_copy(
            vals_v.at[pl.ds(0, P)],
            out_hbm.at[t, pl.ds(pb, P)], zsem)
        for t in range(T)
    ]
    for c in zcopies:
        c.start()

    pltpu.sync_copy(idx_hbm.at[pl.ds(base, CHUNK)], idx_v)
    pltpu.sync_copy(t_hbm.at[pl.ds(base, CHUNK)], t_v)
    pltpu.sync_copy(dt_hbm.at[pl.ds(base, CHUNK)], dt_v)
    pltpu.sync_copy(tef_hbm.at[pl.ds(base * T, CHUNK * T)], tef_v)

    iota = lax.broadcasted_iota(jnp.int32, (L,), 0)
    for j in range(CHUNK // L):
        sl = pl.ds(j * L, L)
        lane = iota + j * L
        idx16 = idx_v[sl]
        te16 = plsc.load_gather(tef_v, [lane * T + idx16])
        th = (te16 - t_v[sl]) / dt_v[sl]
        th_v[sl] = jnp.minimum(jnp.maximum(th, 0.0), 1.0)
        pidx16 = plsc.load_gather(idx_v, [lane ^ 1])
        eq_v[sl] = jnp.where(idx16 == pidx16, 1.0, 0.0)

    for c in zcopies:
        c.wait()

    zi = jnp.zeros((L,), jnp.int32)
    for h in range(2):
        pltpu.sync_copy(y_hbm.at[pl.ds(pb + h * HB, HB)], yh_v)
        pltpu.sync_copy(yn_hbm.at[pl.ds(pb + h * HB, HB)], ynh_v)

        def fbody(jl, _):
            j = h * HB + jl
            tha = plsc.load_gather(th_v, [zi + 2 * j])
            thb = plsc.load_gather(th_v, [zi + 2 * j + 1])
            eqv = plsc.load_gather(eq_v, [zi + 2 * j])
            for k in range(W // L):
                sl = pl.ds(k * L, L)
                th16 = tha if k < D // L else thb
                yv = yh_v[jl, sl]
                ynv = ynh_v[jl, sl]
                v = yv + th16 * (ynv - yv)
                ev = eqv * v
                vals_v[2 * j, sl] = v if k < D // L else ev
                vals_v[2 * j + 1, sl] = ev if k < D // L else v
            return 0
        lax.fori_loop(0, HB, fbody, 0)

    def sbody(j, _):
        idx16 = idx_v[pl.ds(j * L, L)]
        for l in range(L):
            i = j * L + l
            pltpu.make_async_copy(
                vals_v.at[pl.ds(i, 1)],
                out_hbm.at[idx16[l], pl.ds(pb + i // 2, 1)], ssem).start()
        return 0
    lax.fori_loop(0, CHUNK // L, sbody, 0)

    pltpu.make_async_copy(
        y_hbm.at[pl.ds(0, CHUNK)], vals_v, ssem).wait()


def kernel(y_eval, t_eval, t, dt, y, y_next, eval_t_idx, sample_idx):
    T, B, D = y_eval.shape
    CHUNK = B // NW
    mesh = plsc.VectorSubcoreMesh(
        core_axis_name="c", subcore_axis_name="s",
        num_cores=NC, num_subcores=NS)

    k = functools.partial(
        pl.kernel,
        out_type=jax.ShapeDtypeStruct((T, B // 2, 2 * D), jnp.float32),
        mesh=mesh,
        scratch_types=[
            pltpu.VMEM((CHUNK,), jnp.int32),
            pltpu.VMEM((CHUNK,), jnp.float32),
            pltpu.VMEM((CHUNK,), jnp.float32),
            pltpu.VMEM((CHUNK,), jnp.float32),
            pltpu.VMEM((CHUNK,), jnp.float32),
            pltpu.VMEM((CHUNK * T,), jnp.float32),
            pltpu.VMEM((CHUNK // 4, 2 * D), jnp.float32),
            pltpu.VMEM((CHUNK // 4, 2 * D), jnp.float32),
            pltpu.VMEM((CHUNK, 2 * D), jnp.float32),
            pltpu.SemaphoreType.DMA,
            pltpu.SemaphoreType.DMA,
        ],
        compiler_params=pltpu.CompilerParams(needs_layout_passes=False),
    )(functools.partial(_sc_body, T, B, D, CHUNK))

    out = k(t_eval.reshape(B * T), t, dt,
            y.reshape(B // 2, 2 * D), y_next.reshape(B // 2, 2 * D),
            eval_t_idx)
    return out.reshape(T, B, D)


# speedup vs baseline: 1.0358x; 1.0358x over previous
import functools

import jax
import jax.numpy as jnp
from jax import lax
from jax.experimental import pallas as pl
from jax.experimental.pallas import tpu as pltpu
from jax.experimental.pallas import tpu_sc as plsc

NC, NS, L = 2, 16, 16
NW = NC * NS


def _sc_body(T, B, D, CHUNK,
             tef_hbm, t_hbm, dt_hbm, y_hbm, yn_hbm, idx_hbm, out_hbm,
             idx_v, t_v, dt_v, th_v, tef_v, yn_v, zbuf_v, vals_v,
             zsem, ssem):
    wid = lax.axis_index("s") * NC + lax.axis_index("c")
    base = pl.multiple_of(wid * CHUNK, CHUNK)

    zrow = jnp.zeros((L,), jnp.float32)

    def zbody(i, _):
        for k in range(D // L):
            zbuf_v[i, pl.ds(k * L, L)] = zrow
        return 0
    lax.fori_loop(0, CHUNK, zbody, 0)

    zcopies = [
        pltpu.make_async_copy(
            zbuf_v, out_hbm.at[pl.ds(t * B + base, CHUNK)], zsem)
        for t in range(T)
    ]
    for c in zcopies:
        c.start()

    pltpu.sync_copy(idx_hbm.at[pl.ds(base, CHUNK)], idx_v)
    pltpu.sync_copy(t_hbm.at[pl.ds(base, CHUNK)], t_v)
    pltpu.sync_copy(dt_hbm.at[pl.ds(base, CHUNK)], dt_v)
    pltpu.sync_copy(tef_hbm.at[pl.ds(base * T, CHUNK * T)], tef_v)
    pltpu.sync_copy(yn_hbm.at[pl.ds(base, CHUNK)], yn_v)
    pltpu.sync_copy(y_hbm.at[pl.ds(base, CHUNK)], vals_v)

    iota = lax.broadcasted_iota(jnp.int32, (L,), 0)
    for j in range(CHUNK // L):
        sl = pl.ds(j * L, L)
        fi16 = (iota + j * L) * T + idx_v[sl]
        te16 = plsc.load_gather(tef_v, [fi16])
        th = (te16 - t_v[sl]) / dt_v[sl]
        th_v[sl] = jnp.minimum(jnp.maximum(th, 0.0), 1.0)

    zi = jnp.zeros((L,), jnp.int32)

    def fbody(i, _):
        th16 = plsc.load_gather(th_v, [zi + i])
        for k in range(D // L):
            sl = pl.ds(k * L, L)
            yv = vals_v[i, sl]
            ynv = yn_v[i, sl]
            vals_v[i, sl] = yv + th16 * (ynv - yv)
        return 0
    lax.fori_loop(0, CHUNK, fbody, 0)

    for c in zcopies:
        c.wait()

    def sbody(j, _):
        idx16 = idx_v[pl.ds(j * L, L)]
        for l in range(L):
            i = j * L + l
            r = idx16[l] * B + base + i
            pltpu.make_async_copy(
                vals_v.at[pl.ds(i, 1)],
                out_hbm.at[pl.ds(r, 1)], ssem).start()
        return 0
    lax.fori_loop(0, CHUNK // L, sbody, 0)

    pltpu.make_async_copy(
        y_hbm.at[pl.ds(0, CHUNK)], vals_v, ssem).wait()


def kernel(y_eval, t_eval, t, dt, y, y_next, eval_t_idx, sample_idx):
    T, B, D = y_eval.shape
    CHUNK = B // NW
    mesh = plsc.VectorSubcoreMesh(
        core_axis_name="c", subcore_axis_name="s",
        num_cores=NC, num_subcores=NS)

    k = functools.partial(
        pl.kernel,
        out_type=jax.ShapeDtypeStruct((T * B, D), jnp.float32),
        mesh=mesh,
        scratch_types=[
            pltpu.VMEM((CHUNK,), jnp.int32),
            pltpu.VMEM((CHUNK,), jnp.float32),
            pltpu.VMEM((CHUNK,), jnp.float32),
            pltpu.VMEM((CHUNK,), jnp.float32),
            pltpu.VMEM((CHUNK * T,), jnp.float32),
            pltpu.VMEM((CHUNK, D), jnp.float32),
            pltpu.VMEM((CHUNK, D), jnp.float32),
            pltpu.VMEM((CHUNK, D), jnp.float32),
            pltpu.SemaphoreType.DMA,
            pltpu.SemaphoreType.DMA,
        ],
        compiler_params=pltpu.CompilerParams(
            needs_layout_passes=False, use_tc_tiling_on_sc=False),
    )(functools.partial(_sc_body, T, B, D, CHUNK))

    out = k(t_eval.reshape(B * T), t, dt, y, y_next, eval_t_idx)
    return out.reshape(T, B, D)


# speedup vs baseline: 1.7156x; 1.6563x over previous
import functools

import jax
import jax.numpy as jnp
from jax import lax
from jax.experimental import pallas as pl
from jax.experimental.pallas import tpu as pltpu
from jax.experimental.pallas import tpu_sc as plsc

NC, NS, L = 2, 16, 16
NW = NC * NS


def _sc_body(T, B, D, CHUNK,
             tef_hbm, t_hbm, dt_hbm, y_hbm, yn_hbm, idx_hbm, out_hbm,
             idx_v, t_v, dt_v, th_v, tef_v, yn_v, buf_v,
             zsem, ssem):
    wid = lax.axis_index("s") * NC + lax.axis_index("c")
    base = wid * CHUNK

    pltpu.sync_copy(idx_hbm.at[pl.ds(base, CHUNK)], idx_v)
    pltpu.sync_copy(t_hbm.at[pl.ds(base, CHUNK)], t_v)
    pltpu.sync_copy(dt_hbm.at[pl.ds(base, CHUNK)], dt_v)
    pltpu.sync_copy(tef_hbm.at[pl.ds(base * T, CHUNK * T)], tef_v)
    pltpu.sync_copy(yn_hbm.at[pl.ds(base * D, CHUNK * D)], yn_v)

    zrow = jnp.zeros((L,), jnp.float32)

    def zbody(i, _):
        for k in range(D // L):
            buf_v[i, pl.ds(k * L, L)] = zrow
        return 0
    lax.fori_loop(0, CHUNK, zbody, 0)

    zcopies = [
        pltpu.make_async_copy(
            buf_v, out_hbm.at[pl.ds(t * B + base, CHUNK)], zsem)
        for t in range(T)
    ]
    for c in zcopies:
        c.start()

    iota = lax.broadcasted_iota(jnp.int32, (L,), 0)
    for j in range(CHUNK // L):
        sl = pl.ds(j * L, L)
        fi16 = (iota + j * L) * T + idx_v[sl]
        te16 = plsc.load_gather(tef_v, [fi16])
        th = (te16 - t_v[sl]) / dt_v[sl]
        th_v[sl] = jnp.minimum(jnp.maximum(th, 0.0), 1.0)

    for c in zcopies:
        c.wait()

    pltpu.sync_copy(y_hbm.at[pl.ds(base, CHUNK)], buf_v)

    def fbody(i, _):
        th16 = plsc.load_gather(th_v, [jnp.zeros((L,), jnp.int32) + i])
        for k in range(D // L):
            sl = pl.ds(k * L, L)
            yv = buf_v[i, sl]
            ynv = yn_v[pl.ds(i * D + k * L, L)]
            buf_v[i, sl] = yv + th16 * (ynv - yv)
        return 0
    lax.fori_loop(0, CHUNK, fbody, 0)

    def sbody(j, _):
        idx16 = idx_v[pl.ds(j * L, L)]
        for l in range(L):
            i = j * L + l
            r = idx16[l] * B + base + i
            pltpu.make_async_copy(
                buf_v.at[pl.ds(i, 1)], out_hbm.at[pl.ds(r, 1)], ssem).start()
        return 0
    lax.fori_loop(0, CHUNK // L, sbody, 0)

    pltpu.make_async_copy(
        y_hbm.at[pl.ds(0, CHUNK)], buf_v, ssem).wait()


def kernel(y_eval, t_eval, t, dt, y, y_next, eval_t_idx, sample_idx):
    T, B, D = y_eval.shape
    CHUNK = B // NW
    mesh = plsc.VectorSubcoreMesh(
        core_axis_name="c", subcore_axis_name="s",
        num_cores=NC, num_subcores=NS)

    k = functools.partial(
        pl.kernel,
        out_type=jax.ShapeDtypeStruct((T * B, D), jnp.float32),
        mesh=mesh,
        scratch_types=[
            pltpu.VMEM((CHUNK,), jnp.int32),
            pltpu.VMEM((CHUNK,), jnp.float32),
            pltpu.VMEM((CHUNK,), jnp.float32),
            pltpu.VMEM((CHUNK,), jnp.float32),
            pltpu.VMEM((CHUNK * T,), jnp.float32),
            pltpu.VMEM((CHUNK * D,), jnp.float32),
            pltpu.VMEM((CHUNK, D), jnp.float32),
            pltpu.SemaphoreType.DMA,
            pltpu.SemaphoreType.DMA,
        ],
        compiler_params=pltpu.CompilerParams(needs_layout_passes=False),
    )(functools.partial(_sc_body, T, B, D, CHUNK))

    out = k(t_eval.reshape(B * T), t, dt, y,
            y_next.reshape(B * D), eval_t_idx)
    return out.reshape(T, B, D)


# speedup vs baseline: 1.7230x; 1.0043x over previous
import functools

import jax
import jax.numpy as jnp
from jax import lax
from jax.experimental import pallas as pl
from jax.experimental.pallas import tpu as pltpu
from jax.experimental.pallas import tpu_sc as plsc

NC, NS, L = 2, 16, 16
NW = NC * NS


def _sc_body(T, B, D, CHUNK,
             tef_hbm, t_hbm, dt_hbm, y_hbm, yn_hbm, idx_hbm, out_hbm,
             idx_v, t_v, dt_v, th_v, tef_v, yn_v, buf_v,
             zsem, ssem):
    wid = lax.axis_index("s") * NC + lax.axis_index("c")
    base = wid * CHUNK

    zrow = jnp.zeros((L,), jnp.float32)

    def zbody(i, _):
        for k in range(D // L):
            buf_v[i, pl.ds(k * L, L)] = zrow
        return 0
    lax.fori_loop(0, CHUNK, zbody, 0)

    zcopies = [
        pltpu.make_async_copy(
            buf_v, out_hbm.at[pl.ds(t * B + base, CHUNK)], zsem)
        for t in range(T)
    ]
    for c in zcopies:
        c.start()

    pltpu.sync_copy(idx_hbm.at[pl.ds(base, CHUNK)], idx_v)
    pltpu.sync_copy(t_hbm.at[pl.ds(base, CHUNK)], t_v)
    pltpu.sync_copy(dt_hbm.at[pl.ds(base, CHUNK)], dt_v)
    pltpu.sync_copy(tef_hbm.at[pl.ds(base * T, CHUNK * T)], tef_v)
    pltpu.sync_copy(yn_hbm.at[pl.ds(base * D, CHUNK * D)], yn_v)

    iota = lax.broadcasted_iota(jnp.int32, (L,), 0)
    for j in range(CHUNK // L):
        sl = pl.ds(j * L, L)
        fi16 = (iota + j * L) * T + idx_v[sl]
        te16 = plsc.load_gather(tef_v, [fi16])
        th = (te16 - t_v[sl]) / dt_v[sl]
        th_v[sl] = jnp.minimum(jnp.maximum(th, 0.0), 1.0)

    for c in zcopies:
        c.wait()

    pltpu.sync_copy(y_hbm.at[pl.ds(base, CHUNK)], buf_v)

    def fbody(i, _):
        th16 = plsc.load_gather(th_v, [jnp.zeros((L,), jnp.int32) + i])
        for k in range(D // L):
            sl = pl.ds(k * L, L)
            yv = buf_v[i, sl]
            ynv = yn_v[pl.ds(i * D + k * L, L)]
            buf_v[i, sl] = yv + th16 * (ynv - yv)
        return 0
    lax.fori_loop(0, CHUNK, fbody, 0)

    def sbody(j, _):
        idx16 = idx_v[pl.ds(j * L, L)]
        for l in range(L):
            i = j * L + l
            r = idx16[l] * B + base + i
            pltpu.make_async_copy(
                buf_v.at[pl.ds(i, 1)], out_hbm.at[pl.ds(r, 1)], ssem).start()
        return 0
    lax.fori_loop(0, CHUNK // L, sbody, 0)

    pltpu.make_async_copy(
        y_hbm.at[pl.ds(0, CHUNK)], buf_v, ssem).wait()


def kernel(y_eval, t_eval, t, dt, y, y_next, eval_t_idx, sample_idx):
    T, B, D = y_eval.shape
    CHUNK = B // NW
    mesh = plsc.VectorSubcoreMesh(
        core_axis_name="c", subcore_axis_name="s",
        num_cores=NC, num_subcores=NS)

    k = functools.partial(
        pl.kernel,
        out_type=jax.ShapeDtypeStruct((T * B, D), jnp.float32),
        mesh=mesh,
        scratch_types=[
            pltpu.VMEM((CHUNK,), jnp.int32),
            pltpu.VMEM((CHUNK,), jnp.float32),
            pltpu.VMEM((CHUNK,), jnp.float32),
            pltpu.VMEM((CHUNK,), jnp.float32),
            pltpu.VMEM((CHUNK * T,), jnp.float32),
            pltpu.VMEM((CHUNK * D,), jnp.float32),
            pltpu.VMEM((CHUNK, D), jnp.float32),
            pltpu.SemaphoreType.DMA,
            pltpu.SemaphoreType.DMA,
        ],
        compiler_params=pltpu.CompilerParams(needs_layout_passes=False),
    )(functools.partial(_sc_body, T, B, D, CHUNK))

    out = k(t_eval.reshape(B * T), t, dt, y,
            y_next.reshape(B * D), eval_t_idx)
    return out.reshape(T, B, D)
